# Initial kernel scaffold; baseline (speedup 1.0000x reference)
import jax
import jax.numpy as jnp
from jax.experimental import pallas as pl


def kernel(pred, target):
    raise NotImplementedError("write your pallas kernel here")


# speedup vs baseline: 4.2019x; 4.2019x over previous
import jax
import jax.numpy as jnp
from jax.experimental import pallas as pl

IGNORE_INDEX = 255
THRESHOLD = 0.7
MIN_KEPT = 100000

_BLOCK_W = 4096


def _ce_body(pred_ref, tgt_ref, loss_ref, stats_ref):
    x = pred_ref[0]
    t = tgt_ref[0]
    c = x.shape[0]
    m = jnp.max(x, axis=0, keepdims=True)
    s = jnp.sum(jnp.exp(x - m), axis=0, keepdims=True)
    lse = jnp.log(s) + m
    cls = jax.lax.broadcasted_iota(jnp.int32, x.shape, 0)
    xt = jnp.sum(jnp.where(cls == t, x, 0.0), axis=0, keepdims=True)
    valid = t != IGNORE_INDEX
    loss = jnp.where(valid, lse - xt, -1.0)
    loss_ref[0] = loss
    hard = loss > THRESHOLD
    nv = jnp.sum(valid.astype(jnp.float32))
    nh = jnp.sum(hard.astype(jnp.float32))
    hs = jnp.sum(jnp.where(hard, loss, 0.0))
    stats_ref[0] = jnp.concatenate(
        [jnp.full((1, 128), nv, jnp.float32),
         jnp.full((1, 128), nh, jnp.float32),
         jnp.full((1, 128), hs, jnp.float32)], axis=0)


def _topk_mean(loss_flat, num_valid):
    masked = jnp.where(loss_flat >= 0.0, loss_flat, -jnp.inf)
    k_static = min(MIN_KEPT, loss_flat.size)
    vals, _ = jax.lax.top_k(masked, k_static)
    k_eff = jnp.minimum(jnp.float32(MIN_KEPT), num_valid)
    keep = jnp.arange(k_static, dtype=jnp.float32) < k_eff
    s = jnp.sum(jnp.where(keep, vals, 0.0))
    return s / jnp.maximum(k_eff, 1.0)


def kernel(pred, target):
    b, c, h, w = pred.shape
    hw = h * w
    pred3 = pred.reshape(b, c, hw)
    tgt3 = target.reshape(b, 1, hw)
    bw = min(_BLOCK_W, hw)
    np_ = hw // bw
    grid = (b, np_)
    loss3, stats = pl.pallas_call(
        _ce_body,
        grid=grid,
        in_specs=[
            pl.BlockSpec((1, c, bw), lambda i, j: (i, 0, j)),
            pl.BlockSpec((1, 1, bw), lambda i, j: (i, 0, j)),
        ],
        out_specs=[
            pl.BlockSpec((1, 1, bw), lambda i, j: (i, 0, j)),
            pl.BlockSpec((1, 3, 128), lambda i, j: (i * np_ + j, 0, 0)),
        ],
        out_shape=[
            jax.ShapeDtypeStruct((b, 1, hw), jnp.float32),
            jax.ShapeDtypeStruct((b * np_, 3, 128), jnp.float32),
        ],
    )(pred3, tgt3)
    loss_flat = loss3.reshape(-1)
    num_valid = jnp.sum(stats[:, 0, 0])
    num_hard = jnp.sum(stats[:, 1, 0])
    hard_sum = jnp.sum(stats[:, 2, 0])
    out = jax.lax.cond(
        num_hard < MIN_KEPT,
        lambda: _topk_mean(loss_flat, num_valid),
        lambda: hard_sum / jnp.maximum(num_hard, 1.0),
    )
    return jnp.where(num_valid == 0.0, jnp.float32(0.0), out)


# speedup vs baseline: 6.0498x; 1.4398x over previous
import jax
import jax.numpy as jnp
from jax.experimental import pallas as pl

IGNORE_INDEX = 255
THRESHOLD = 0.7
MIN_KEPT = 100000

_BLOCK_W = 16384


def _ce_body(pred_ref, tgt_ref, loss_ref, stats_ref):
    x = pred_ref[0]
    t = tgt_ref[0]
    c = x.shape[0]
    m = jnp.max(x, axis=0, keepdims=True)
    s = jnp.sum(jnp.exp(x - m), axis=0, keepdims=True)
    lse = jnp.log(s) + m
    cls = jax.lax.broadcasted_iota(jnp.int32, x.shape, 0)
    xt = jnp.sum(jnp.where(cls == t, x, 0.0), axis=0, keepdims=True)
    valid = t != IGNORE_INDEX
    loss = jnp.where(valid, lse - xt, -1.0)
    loss_ref[0] = loss
    hard = loss > THRESHOLD
    nv = jnp.sum(valid.astype(jnp.float32))
    nh = jnp.sum(hard.astype(jnp.float32))
    hs = jnp.sum(jnp.where(hard, loss, 0.0))
    stats_ref[0] = jnp.concatenate(
        [jnp.full((1, 128), nv, jnp.float32),
         jnp.full((1, 128), nh, jnp.float32),
         jnp.full((1, 128), hs, jnp.float32)], axis=0)


def _topk_mean(loss_flat, num_valid):
    masked = jnp.where(loss_flat >= 0.0, loss_flat, -jnp.inf)
    k_static = min(MIN_KEPT, loss_flat.size)
    vals, _ = jax.lax.top_k(masked, k_static)
    k_eff = jnp.minimum(jnp.float32(MIN_KEPT), num_valid)
    keep = jnp.arange(k_static, dtype=jnp.float32) < k_eff
    s = jnp.sum(jnp.where(keep, vals, 0.0))
    return s / jnp.maximum(k_eff, 1.0)


def kernel(pred, target):
    b, c, h, w = pred.shape
    hw = h * w
    pred3 = pred.reshape(b, c, hw)
    tgt3 = target.reshape(b, 1, hw)
    bw = min(_BLOCK_W, hw)
    np_ = hw // bw
    grid = (b, np_)
    loss3, stats = pl.pallas_call(
        _ce_body,
        grid=grid,
        in_specs=[
            pl.BlockSpec((1, c, bw), lambda i, j: (i, 0, j)),
            pl.BlockSpec((1, 1, bw), lambda i, j: (i, 0, j)),
        ],
        out_specs=[
            pl.BlockSpec((1, 1, bw), lambda i, j: (i, 0, j)),
            pl.BlockSpec((1, 3, 128), lambda i, j: (i * np_ + j, 0, 0)),
        ],
        out_shape=[
            jax.ShapeDtypeStruct((b, 1, hw), jnp.float32),
            jax.ShapeDtypeStruct((b * np_, 3, 128), jnp.float32),
        ],
    )(pred3, tgt3)
    loss_flat = loss3.reshape(-1)
    num_valid = jnp.sum(stats[:, 0, 0])
    num_hard = jnp.sum(stats[:, 1, 0])
    hard_sum = jnp.sum(stats[:, 2, 0])
    out = jax.lax.cond(
        num_hard < MIN_KEPT,
        lambda: _topk_mean(loss_flat, num_valid),
        lambda: hard_sum / jnp.maximum(num_hard, 1.0),
    )
    return jnp.where(num_valid == 0.0, jnp.float32(0.0), out)


# speedup vs baseline: 6.3056x; 1.0423x over previous
import jax
import jax.numpy as jnp
from jax.experimental import pallas as pl

IGNORE_INDEX = 255
THRESHOLD = 0.7
MIN_KEPT = 100000

_BLOCK_W = 32768


def _ce_body(pred_ref, tgt_ref, loss_ref, stats_ref):
    x = pred_ref[0]
    t = tgt_ref[0]
    c = x.shape[0]
    m = jnp.max(x, axis=0, keepdims=True)
    s = jnp.sum(jnp.exp(x - m), axis=0, keepdims=True)
    lse = jnp.log(s) + m
    cls = jax.lax.broadcasted_iota(jnp.int32, x.shape, 0)
    xt = jnp.sum(jnp.where(cls == t, x, 0.0), axis=0, keepdims=True)
    valid = t != IGNORE_INDEX
    loss = jnp.where(valid, lse - xt, -1.0)
    loss_ref[0] = loss
    hard = loss > THRESHOLD
    nv = jnp.sum(valid.astype(jnp.float32))
    nh = jnp.sum(hard.astype(jnp.float32))
    hs = jnp.sum(jnp.where(hard, loss, 0.0))
    stats_ref[0] = jnp.concatenate(
        [jnp.full((1, 128), nv, jnp.float32),
         jnp.full((1, 128), nh, jnp.float32),
         jnp.full((1, 128), hs, jnp.float32)], axis=0)


def _topk_mean(loss_flat, num_valid):
    masked = jnp.where(loss_flat >= 0.0, loss_flat, -jnp.inf)
    k_static = min(MIN_KEPT, loss_flat.size)
    vals, _ = jax.lax.top_k(masked, k_static)
    k_eff = jnp.minimum(jnp.float32(MIN_KEPT), num_valid)
    keep = jnp.arange(k_static, dtype=jnp.float32) < k_eff
    s = jnp.sum(jnp.where(keep, vals, 0.0))
    return s / jnp.maximum(k_eff, 1.0)


def kernel(pred, target):
    b, c, h, w = pred.shape
    hw = h * w
    pred3 = pred.reshape(b, c, hw)
    tgt3 = target.reshape(b, 1, hw)
    bw = min(_BLOCK_W, hw)
    np_ = hw // bw
    grid = (b, np_)
    loss3, stats = pl.pallas_call(
        _ce_body,
        grid=grid,
        in_specs=[
            pl.BlockSpec((1, c, bw), lambda i, j: (i, 0, j)),
            pl.BlockSpec((1, 1, bw), lambda i, j: (i, 0, j)),
        ],
        out_specs=[
            pl.BlockSpec((1, 1, bw), lambda i, j: (i, 0, j)),
            pl.BlockSpec((1, 3, 128), lambda i, j: (i * np_ + j, 0, 0)),
        ],
        out_shape=[
            jax.ShapeDtypeStruct((b, 1, hw), jnp.float32),
            jax.ShapeDtypeStruct((b * np_, 3, 128), jnp.float32),
        ],
    )(pred3, tgt3)
    loss_flat = loss3.reshape(-1)
    num_valid = jnp.sum(stats[:, 0, 0])
    num_hard = jnp.sum(stats[:, 1, 0])
    hard_sum = jnp.sum(stats[:, 2, 0])
    out = jax.lax.cond(
        num_hard < MIN_KEPT,
        lambda: _topk_mean(loss_flat, num_valid),
        lambda: hard_sum / jnp.maximum(num_hard, 1.0),
    )
    return jnp.where(num_valid == 0.0, jnp.float32(0.0), out)


# speedup vs baseline: 22.0209x; 3.4923x over previous
import jax
import jax.numpy as jnp
from jax.experimental import pallas as pl

IGNORE_INDEX = 255
THRESHOLD = 0.7
MIN_KEPT = 100000

_BLOCK_H = 64


def _ce_body(pred_ref, tgt_ref, loss_ref, stats_ref):
    x = pred_ref[0]
    t = tgt_ref[0]
    m = jnp.max(x, axis=0)
    s = jnp.sum(jnp.exp(x - m[None]), axis=0)
    lse = jnp.log(s) + m
    cls = jax.lax.broadcasted_iota(jnp.int32, x.shape, 0)
    xt = jnp.sum(jnp.where(cls == t[None], x, 0.0), axis=0)
    valid = t != IGNORE_INDEX
    loss = jnp.where(valid, lse - xt, -1.0)
    loss_ref[0] = loss
    hard = loss > THRESHOLD
    nv = jnp.sum(valid.astype(jnp.float32))
    nh = jnp.sum(hard.astype(jnp.float32))
    hs = jnp.sum(jnp.where(hard, loss, 0.0))
    stats_ref[0] = jnp.concatenate(
        [jnp.full((1, 128), nv, jnp.float32),
         jnp.full((1, 128), nh, jnp.float32),
         jnp.full((1, 128), hs, jnp.float32)], axis=0)


def _topk_mean(loss3, num_valid):
    loss_flat = loss3.reshape(-1)
    masked = jnp.where(loss_flat >= 0.0, loss_flat, -jnp.inf)
    k_static = min(MIN_KEPT, loss_flat.size)
    vals, _ = jax.lax.top_k(masked, k_static)
    k_eff = jnp.minimum(jnp.float32(MIN_KEPT), num_valid)
    keep = jnp.arange(k_static, dtype=jnp.float32) < k_eff
    s = jnp.sum(jnp.where(keep, vals, 0.0))
    return s / jnp.maximum(k_eff, 1.0)


def kernel(pred, target):
    b, c, h, w = pred.shape
    hb = min(_BLOCK_H, h)
    nh_blocks = h // hb
    grid = (b, nh_blocks)
    loss3, stats = pl.pallas_call(
        _ce_body,
        grid=grid,
        in_specs=[
            pl.BlockSpec((1, c, hb, w), lambda i, j: (i, 0, j, 0)),
            pl.BlockSpec((1, hb, w), lambda i, j: (i, j, 0)),
        ],
        out_specs=[
            pl.BlockSpec((1, hb, w), lambda i, j: (i, j, 0)),
            pl.BlockSpec((1, 3, 128), lambda i, j: (i * nh_blocks + j, 0, 0)),
        ],
        out_shape=[
            jax.ShapeDtypeStruct((b, h, w), jnp.float32),
            jax.ShapeDtypeStruct((b * nh_blocks, 3, 128), jnp.float32),
        ],
    )(pred, target)
    num_valid = jnp.sum(stats[:, 0, 0])
    num_hard = jnp.sum(stats[:, 1, 0])
    hard_sum = jnp.sum(stats[:, 2, 0])
    out = jax.lax.cond(
        num_hard < MIN_KEPT,
        lambda: _topk_mean(loss3, num_valid),
        lambda: hard_sum / jnp.maximum(num_hard, 1.0),
    )
    return jnp.where(num_valid == 0.0, jnp.float32(0.0), out)


# speedup vs baseline: 25.2577x; 1.1470x over previous
import jax
import jax.numpy as jnp
from jax.experimental import pallas as pl

IGNORE_INDEX = 255
THRESHOLD = 0.7
MIN_KEPT = 100000

_BLOCK_H = 128


def _ce_body(pred_ref, tgt_ref, loss_ref, stats_ref):
    x = pred_ref[0]
    t = tgt_ref[0]
    m = jnp.max(x, axis=0)
    s = jnp.sum(jnp.exp(x - m[None]), axis=0)
    lse = jnp.log(s) + m
    cls = jax.lax.broadcasted_iota(jnp.int32, x.shape, 0)
    xt = jnp.sum(jnp.where(cls == t[None], x, 0.0), axis=0)
    valid = t != IGNORE_INDEX
    loss = jnp.where(valid, lse - xt, -1.0)
    loss_ref[0] = loss
    hard = loss > THRESHOLD
    nv = jnp.sum(valid.astype(jnp.float32))
    nh = jnp.sum(hard.astype(jnp.float32))
    hs = jnp.sum(jnp.where(hard, loss, 0.0))
    stats_ref[0] = jnp.concatenate(
        [jnp.full((1, 128), nv, jnp.float32),
         jnp.full((1, 128), nh, jnp.float32),
         jnp.full((1, 128), hs, jnp.float32)], axis=0)


def _topk_mean(loss3, num_valid):
    loss_flat = loss3.reshape(-1)
    masked = jnp.where(loss_flat >= 0.0, loss_flat, -jnp.inf)
    k_static = min(MIN_KEPT, loss_flat.size)
    vals, _ = jax.lax.top_k(masked, k_static)
    k_eff = jnp.minimum(jnp.float32(MIN_KEPT), num_valid)
    keep = jnp.arange(k_static, dtype=jnp.float32) < k_eff
    s = jnp.sum(jnp.where(keep, vals, 0.0))
    return s / jnp.maximum(k_eff, 1.0)


def kernel(pred, target):
    b, c, h, w = pred.shape
    hb = min(_BLOCK_H, h)
    nh_blocks = h // hb
    grid = (b, nh_blocks)
    loss3, stats = pl.pallas_call(
        _ce_body,
        grid=grid,
        in_specs=[
            pl.BlockSpec((1, c, hb, w), lambda i, j: (i, 0, j, 0)),
            pl.BlockSpec((1, hb, w), lambda i, j: (i, j, 0)),
        ],
        out_specs=[
            pl.BlockSpec((1, hb, w), lambda i, j: (i, j, 0)),
            pl.BlockSpec((1, 3, 128), lambda i, j: (i * nh_blocks + j, 0, 0)),
        ],
        out_shape=[
            jax.ShapeDtypeStruct((b, h, w), jnp.float32),
            jax.ShapeDtypeStruct((b * nh_blocks, 3, 128), jnp.float32),
        ],
    )(pred, target)
    num_valid = jnp.sum(stats[:, 0, 0])
    num_hard = jnp.sum(stats[:, 1, 0])
    hard_sum = jnp.sum(stats[:, 2, 0])
    out = jax.lax.cond(
        num_hard < MIN_KEPT,
        lambda: _topk_mean(loss3, num_valid),
        lambda: hard_sum / jnp.maximum(num_hard, 1.0),
    )
    return jnp.where(num_valid == 0.0, jnp.float32(0.0), out)


# speedup vs baseline: 25.5234x; 1.0105x over previous
import jax
import jax.numpy as jnp
from jax.experimental import pallas as pl

IGNORE_INDEX = 255
THRESHOLD = 0.7
MIN_KEPT = 100000

_BLOCK_H = 256


def _ce_body(pred_ref, tgt_ref, loss_ref, stats_ref):
    x = pred_ref[0]
    t = tgt_ref[0]
    m = jnp.max(x, axis=0)
    s = jnp.sum(jnp.exp(x - m[None]), axis=0)
    lse = jnp.log(s) + m
    cls = jax.lax.broadcasted_iota(jnp.int32, x.shape, 0)
    xt = jnp.sum(jnp.where(cls == t[None], x, 0.0), axis=0)
    valid = t != IGNORE_INDEX
    loss = jnp.where(valid, lse - xt, -1.0)
    loss_ref[0] = loss
    hard = loss > THRESHOLD
    nv = jnp.sum(valid.astype(jnp.float32))
    nh = jnp.sum(hard.astype(jnp.float32))
    hs = jnp.sum(jnp.where(hard, loss, 0.0))
    stats_ref[0] = jnp.concatenate(
        [jnp.full((1, 128), nv, jnp.float32),
         jnp.full((1, 128), nh, jnp.float32),
         jnp.full((1, 128), hs, jnp.float32)], axis=0)


def _topk_mean(loss3, num_valid):
    loss_flat = loss3.reshape(-1)
    masked = jnp.where(loss_flat >= 0.0, loss_flat, -jnp.inf)
    k_static = min(MIN_KEPT, loss_flat.size)
    vals, _ = jax.lax.top_k(masked, k_static)
    k_eff = jnp.minimum(jnp.float32(MIN_KEPT), num_valid)
    keep = jnp.arange(k_static, dtype=jnp.float32) < k_eff
    s = jnp.sum(jnp.where(keep, vals, 0.0))
    return s / jnp.maximum(k_eff, 1.0)


def kernel(pred, target):
    b, c, h, w = pred.shape
    hb = min(_BLOCK_H, h)
    nh_blocks = h // hb
    grid = (b, nh_blocks)
    loss3, stats = pl.pallas_call(
        _ce_body,
        grid=grid,
        in_specs=[
            pl.BlockSpec((1, c, hb, w), lambda i, j: (i, 0, j, 0)),
            pl.BlockSpec((1, hb, w), lambda i, j: (i, j, 0)),
        ],
        out_specs=[
            pl.BlockSpec((1, hb, w), lambda i, j: (i, j, 0)),
            pl.BlockSpec((1, 3, 128), lambda i, j: (i * nh_blocks + j, 0, 0)),
        ],
        out_shape=[
            jax.ShapeDtypeStruct((b, h, w), jnp.float32),
            jax.ShapeDtypeStruct((b * nh_blocks, 3, 128), jnp.float32),
        ],
    )(pred, target)
    num_valid = jnp.sum(stats[:, 0, 0])
    num_hard = jnp.sum(stats[:, 1, 0])
    hard_sum = jnp.sum(stats[:, 2, 0])
    out = jax.lax.cond(
        num_hard < MIN_KEPT,
        lambda: _topk_mean(loss3, num_valid),
        lambda: hard_sum / jnp.maximum(num_hard, 1.0),
    )
    return jnp.where(num_valid == 0.0, jnp.float32(0.0), out)
